# baseline (speedup 1.0000x reference)
import jax
import jax.numpy as jnp
from jax.experimental import pallas as pl
from jax.experimental.pallas import tpu as pltpu

B, K, N = 256, 8, 128
G = 4
EPS = 1e-12


def _loss_kernel(nn_ref, x_ref, out_ref):
    bb = pl.program_id(0)
    dims_i = (((1,), (1,)), ((), ()))
    dims_bc = (((2,), (0,)), ((), ()))
    ones_b = jnp.ones((N, N), dtype=jnp.bfloat16)

    for g in range(G):
        m = nn_ref[bb * G + g]
        x = x_ref[g]

        xc = jnp.maximum(x, EPS)
        xcb = xc.astype(jnp.bfloat16)

        row2 = jax.lax.broadcasted_iota(jnp.int32, (2, N), 0)
        col2 = jax.lax.broadcasted_iota(jnp.int32, (2, N), 1)
        wt = jnp.where((row2 == 0) | (col2 < m), 1.0, 0.0).astype(jnp.bfloat16)

        cl = jax.lax.dot_general(wt, xcb, dims_i,
                                 preferred_element_type=jnp.float32)
        c = jnp.maximum(cl[0], EPS)
        d = cl[1]
        cinv = 1.0 / c

        rb = jax.lax.dot_general(xcb, ones_b, dims_bc,
                                 preferred_element_type=jnp.float32)

        def _half(lo):
            xch = xc[:, lo:lo + N // 2, :]
            rbh = jnp.maximum(rb[:, lo:lo + N // 2, :], EPS)
            lh = xch * jnp.log(xch)
            qh = xch * jax.lax.reciprocal(rbh)
            imh = lo + jax.lax.broadcasted_iota(
                jnp.int32, (K, N // 2, N), 1) < m
            hh = jnp.where(imh, qh * jnp.log(qh) + lh * cinv[:, None, :], 0.0)
            return jnp.sum(hh, axis=1)

        hc = _half(0) + jax.lax.cond(
            m > N // 2, lambda: _half(N // 2),
            lambda: jnp.zeros((K, N), jnp.float32))

        vm = jax.lax.broadcasted_iota(jnp.int32, (K, N), 1) < m
        e_colpart = d * jnp.log(c) * cinv
        loss_b = jnp.sum(jnp.where(vm, e_colpart - hc, 0.0))
        out_ref[g, 0, 0] = loss_b / (K * m.astype(jnp.float32))


def kernel(perm_soft, n_nodes):
    nn = n_nodes.astype(jnp.int32)
    out = pl.pallas_call(
        _loss_kernel,
        grid_spec=pltpu.PrefetchScalarGridSpec(
            num_scalar_prefetch=1,
            grid=(B // G,),
            in_specs=[
                pl.BlockSpec((G, K, N, N), lambda b, nn_ref: (b, 0, 0, 0)),
            ],
            out_specs=pl.BlockSpec(
                (G, 1, 1), lambda b, nn_ref: (b, 0, 0), memory_space=pltpu.SMEM
            ),
        ),
        out_shape=jax.ShapeDtypeStruct((B, 1, 1), jnp.float32),
    )(nn, perm_soft)
    return jnp.mean(out)


# speedup vs baseline: 1.2404x; 1.2404x over previous
import jax
import jax.numpy as jnp
from jax.experimental import pallas as pl
from jax.experimental.pallas import tpu as pltpu

B, K, N = 256, 8, 128
G = 4
EPS = 1e-12


def _loss_kernel(nn_ref, x_ref, out_ref):
    bb = pl.program_id(0)
    dims_i = (((1,), (1,)), ((), ()))
    dims_bc = (((2,), (0,)), ((), ()))
    ones_b = jnp.ones((N, N), dtype=jnp.bfloat16)

    for g in range(G):
        m = nn_ref[bb * G + g]
        x = x_ref[g]

        xc = jnp.maximum(x, EPS)
        xcb = xc.astype(jnp.bfloat16)

        row2 = jax.lax.broadcasted_iota(jnp.int32, (2, N), 0)
        col2 = jax.lax.broadcasted_iota(jnp.int32, (2, N), 1)
        wt = jnp.where((row2 == 0) | (col2 < m), 1.0, 0.0).astype(jnp.bfloat16)

        cl = jax.lax.dot_general(wt, xcb, dims_i,
                                 preferred_element_type=jnp.float32)
        c = jnp.maximum(cl[0], EPS)
        d = cl[1]
        cinv = 1.0 / c

        l = xc * jnp.log(xc)
        im3 = jax.lax.broadcasted_iota(jnp.int32, (K, N, N), 1) < m
        rb = jax.lax.dot_general(xcb, ones_b, dims_bc,
                                 preferred_element_type=jnp.float32)
        rb = jnp.maximum(rb, EPS)
        q = xc * jax.lax.reciprocal(rb)
        h = jnp.where(im3, q * jnp.log(q) + l * cinv[:, None, :], 0.0)
        hc = jnp.sum(h, axis=1)

        vm = jax.lax.broadcasted_iota(jnp.int32, (K, N), 1) < m
        e_colpart = d * jnp.log(c) * cinv
        loss_b = jnp.sum(jnp.where(vm, e_colpart - hc, 0.0))
        out_ref[g, 0, 0] = loss_b / (K * m.astype(jnp.float32))


def kernel(perm_soft, n_nodes):
    nn = n_nodes.astype(jnp.int32)
    out = pl.pallas_call(
        _loss_kernel,
        grid_spec=pltpu.PrefetchScalarGridSpec(
            num_scalar_prefetch=1,
            grid=(B // G,),
            in_specs=[
                pl.BlockSpec((G, K, N, N), lambda b, nn_ref: (b, 0, 0, 0)),
            ],
            out_specs=pl.BlockSpec(
                (G, 1, 1), lambda b, nn_ref: (b, 0, 0), memory_space=pltpu.SMEM
            ),
        ),
        out_shape=jax.ShapeDtypeStruct((B, 1, 1), jnp.float32),
    )(nn, perm_soft)
    return jnp.mean(out)
